# baseline (speedup 1.0000x reference)
import functools

import jax
import jax.numpy as jnp
from jax import lax
from jax.experimental import pallas as pl
from jax.experimental.pallas import tpu as pltpu
from jax.experimental.pallas import tpu_sc as plsc

_LENGTH = 1024
_VOCAB = 2049
_D = 128
_NC = 2
_NS = 16
_NW = _NC * _NS
_ROWS_PER_W = _LENGTH // _NW
_WIN = 544
_K = 13

_mesh = plsc.VectorSubcoreMesh(core_axis_name="c", subcore_axis_name="s")


@functools.partial(
    pl.kernel,
    mesh=_mesh,
    out_type=jax.ShapeDtypeStruct((_LENGTH, _LENGTH, _D), jnp.float32),
    scratch_types=[
        pltpu.VMEM_SHARED((_VOCAB, _D), jnp.float32),
        pltpu.VMEM((_WIN, _D), jnp.float32),
        pltpu.SemaphoreType.DMA,
        pltpu.SemaphoreType.DMA,
    ],
)
def _sc_copy(table_hbm, out_hbm, table_sp, buf, sp_sem, st_sem):
    c = lax.axis_index("c")
    s = lax.axis_index("s")
    wid = s * _NC + c
    row0 = wid * _ROWS_PER_W
    base = 992 - _ROWS_PER_W * wid

    @pl.when(s == 0)
    def _stage_table():
        pltpu.sync_copy(table_hbm, table_sp)

    plsc.subcore_barrier()

    def _sp_row(t):
        i = row0 + t
        return pltpu.make_async_copy(
            table_sp.at[pl.ds(_LENGTH - i, _LENGTH)], out_hbm.at[i], sp_sem
        )

    def _sp_fire(t, cc):
        _sp_row(t).start()
        return cc

    lax.fori_loop(0, _K, _sp_fire, 0)

    def _st_row(t, p):
        i = row0 + t
        return pltpu.make_async_copy(
            buf.at[pl.ds(_ROWS_PER_W - t, 512)],
            out_hbm.at[i, pl.ds(512 * p, 512)],
            st_sem,
        )

    def _phase(p, carry):
        pltpu.sync_copy(table_hbm.at[pl.ds(base + 512 * p, _WIN)], buf)

        def _fire(t, cc):
            _st_row(t, p).start()
            return cc

        def _drain(t, cc):
            _st_row(t, p).wait()
            return cc

        lax.fori_loop(_K, _ROWS_PER_W, _fire, 0)
        lax.fori_loop(_K, _ROWS_PER_W, _drain, 0)
        return carry

    lax.fori_loop(0, 2, _phase, 0)

    def _sp_drain(t, cc):
        _sp_row(t).wait()
        return cc

    lax.fori_loop(0, _K, _sp_drain, 0)


def kernel(length, embedding_table):
    del length
    return _sc_copy(embedding_table)


# speedup vs baseline: 1.0168x; 1.0168x over previous
import functools

import jax
import jax.numpy as jnp
from jax import lax
from jax.experimental import pallas as pl
from jax.experimental.pallas import tpu as pltpu
from jax.experimental.pallas import tpu_sc as plsc

_LENGTH = 1024
_VOCAB = 2049
_D = 128
_NC = 2
_NS = 16
_NW = _NC * _NS
_ROWS_PER_W = _LENGTH // _NW
_WIN = 544

_mesh = plsc.VectorSubcoreMesh(core_axis_name="c", subcore_axis_name="s")


@functools.partial(
    pl.kernel,
    mesh=_mesh,
    out_type=jax.ShapeDtypeStruct((_LENGTH, _LENGTH, _D), jnp.float32),
    scratch_types=[
        pltpu.VMEM((_WIN, _D), jnp.float32),
        pltpu.SemaphoreType.DMA,
    ],
)
def _sc_copy(table_hbm, out_hbm, buf, sem):
    c = lax.axis_index("c")
    s = lax.axis_index("s")
    wid = s * _NC + c
    base = 992 - _ROWS_PER_W * wid

    def _phase(p, carry):
        start = base + 512 * p
        pltpu.sync_copy(table_hbm.at[pl.ds(start, _WIN)], buf)

        def _row(t, i):
            return pltpu.make_async_copy(
                buf.at[pl.ds(_ROWS_PER_W - t, 512)],
                out_hbm.at[i, pl.ds(512 * p, 512)],
                sem,
            )

        def _fire(t, cc):
            _row(t, wid * _ROWS_PER_W + t).start()
            return cc

        def _drain(t, cc):
            _row(t, wid * _ROWS_PER_W + t).wait()
            return cc

        lax.fori_loop(0, _ROWS_PER_W, _fire, 0)
        lax.fori_loop(0, _ROWS_PER_W, _drain, 0)
        return carry

    lax.fori_loop(0, 2, _phase, 0)


def kernel(length, embedding_table):
    del length
    return _sc_copy(embedding_table)
